# baseline (speedup 1.0000x reference)
import functools

import jax
import jax.numpy as jnp
import numpy as np
from jax import lax
from jax.experimental import pallas as pl
from jax.experimental.pallas import tpu as pltpu
from jax.experimental.pallas import tpu_sc as plsc


def _build_sc_kernel(b, h, w, d):
    mesh = plsc.VectorSubcoreMesh(core_axis_name="c", subcore_axis_name="s")

    @functools.partial(
        pl.kernel,
        mesh=mesh,
        out_type=jax.ShapeDtypeStruct((b, h * w, 2 * d), jnp.float32),
        scratch_types=[
            pltpu.VMEM((w, 2 * d), jnp.float32),
            pltpu.VMEM((w,), jnp.int32),
            pltpu.SemaphoreType.DMA,
        ],
    )
    def sc_kernel(row_hbm, col_hbm, out_hbm, buf, idx, sem):
        cid = lax.axis_index("c")
        sid = lax.axis_index("s")
        i = sid * 2 + cid

        pltpu.sync_copy(col_hbm.at[pl.ds(0, w)], buf.at[:, pl.ds(0, d)])
        for c in range(0, w, 16):
            idx[pl.ds(c, 16)] = jnp.full((16,), i, jnp.int32)
        pltpu.async_copy(row_hbm.at[idx], buf.at[:, pl.ds(d, d)], sem).wait()

        copies = [
            pltpu.async_copy(buf, out_hbm.at[bb, pl.ds(i * w, w)], sem)
            for bb in range(b)
        ]
        for cp in copies:
            cp.wait()

    return sc_kernel


def kernel(x, row_embed, col_embed):
    b = x.shape[0]
    hw = x.shape[1]
    h = w = int(np.sqrt(hw))
    d = row_embed.shape[1]
    return _build_sc_kernel(b, h, w, d)(row_embed, col_embed)


# speedup vs baseline: 1.1237x; 1.1237x over previous
import functools

import jax
import jax.numpy as jnp
import numpy as np
from jax import lax
from jax.experimental import pallas as pl
from jax.experimental.pallas import tpu as pltpu
from jax.experimental.pallas import tpu_sc as plsc


def _build_sc_kernel(b, h, w, d):
    mesh = plsc.VectorSubcoreMesh(core_axis_name="c", subcore_axis_name="s")

    @functools.partial(
        pl.kernel,
        mesh=mesh,
        out_type=jax.ShapeDtypeStruct((b, h * w, 2 * d), jnp.float32),
        scratch_types=[
            pltpu.VMEM((w, d), jnp.float32),
            pltpu.VMEM((d,), jnp.float32),
            pltpu.VMEM((w, 2 * d), jnp.float32),
            pltpu.SemaphoreType.DMA,
        ],
    )
    def sc_kernel(row_hbm, col_hbm, out_hbm, colv, rowv, buf, sem):
        cid = lax.axis_index("c")
        sid = lax.axis_index("s")
        i = sid * 2 + cid

        pltpu.sync_copy(col_hbm.at[pl.ds(0, w)], colv)
        pltpu.sync_copy(row_hbm.at[i], rowv)

        for c in range(d // 16):
            v = rowv[pl.ds(c * 16, 16)]
            for j in range(w):
                buf[j, pl.ds(d + c * 16, 16)] = v
        for j in range(w):
            for c in range(d // 16):
                buf[j, pl.ds(c * 16, 16)] = colv[j, pl.ds(c * 16, 16)]

        copies = [
            pltpu.async_copy(buf, out_hbm.at[bb, pl.ds(i * w, w)], sem)
            for bb in range(b)
        ]
        for cp in copies:
            cp.wait()

    return sc_kernel


def kernel(x, row_embed, col_embed):
    b = x.shape[0]
    hw = x.shape[1]
    h = w = int(np.sqrt(hw))
    d = row_embed.shape[1]
    return _build_sc_kernel(b, h, w, d)(row_embed, col_embed)
